# Initial kernel scaffold; baseline (speedup 1.0000x reference)
import functools

import jax
import jax.numpy as jnp
from jax import lax
from jax.experimental import pallas as pl
from jax.experimental.pallas import tpu as pltpu
from jax.experimental.pallas import tpu_sc as plsc

KCB = 1024
DTOK = 256
NTOK = 9216
NCHUNK = 12
CW = NTOK // NCHUNK
EPS = 10.0
NITER = 5


def _assign_body(x_ref, e_ref, w_ref, b_ref, idx_ref, emb_ref, q_ref, cs_ref,
                 y_ref, y2_ref, smem):
    i = pl.program_id(0)

    @pl.when(i == 0)
    def _init():
        y = lax.dot_general(e_ref[...], w_ref[...],
                            (((1,), (1,)), ((), ())),
                            preferred_element_type=jnp.float32)
        y = y + b_ref[...]
        y_ref[...] = y
        emb_ref[...] = y
        y2_ref[...] = jnp.sum(y * y, axis=1, keepdims=True)
        smem[0] = 0.0
        smem[1] = jnp.inf

    xb = x_ref[...]
    sc = lax.dot_general(y_ref[...], xb, (((1,), (0,)), ((), ())),
                         preferred_element_type=jnp.float32)
    x2 = jnp.sum(xb * xb, axis=0, keepdims=True)
    d2 = (x2 + y2_ref[...]) - 2.0 * sc
    d = jnp.sqrt(jnp.maximum(d2, 0.0))
    q_ref[i] = d
    smem[0] = smem[0] + jnp.sum(d)
    smem[1] = jnp.minimum(smem[1], jnp.min(d))

    @pl.when(i == NCHUNK - 1)
    def _final():
        kn = float(KCB * NTOK)
        mean = smem[0] / kn

        def var_step(c, acc):
            t = q_ref[c] - mean
            return acc + jnp.sum(t * t)
        s2 = lax.fori_loop(0, NCHUNK, var_step, 0.0)
        std = jnp.sqrt(s2 / (kn - 1.0))
        denom = std + 1e-8
        dmin = (smem[1] - mean) / denom

        def exp_step(c, acc):
            dn = (q_ref[c] - mean) / denom - dmin
            q = jnp.exp(-dn * EPS)
            q_ref[c] = q
            return acc + jnp.sum(q)
        s = lax.fori_loop(0, NCHUNK, exp_step, 0.0)

        def sdiv_step(c, rs):
            q = q_ref[c] / (s + 1e-8)
            q_ref[c] = q
            return rs + jnp.sum(q, axis=1, keepdims=True)
        rs = lax.fori_loop(0, NCHUNK, sdiv_step, jnp.zeros((KCB, 1), jnp.float32))

        for it in range(NITER):
            def row_step(c, _):
                q = q_ref[c] / (rs + 1e-8)
                q = q / float(KCB)
                q_ref[c] = q
                cs_ref[c] = jnp.sum(q, axis=0, keepdims=True)
                return 0
            lax.fori_loop(0, NCHUNK, row_step, 0)

            last = it == NITER - 1

            def col_step(c, rs_acc):
                q = q_ref[c] / (cs_ref[c] + 1e-8)
                q = q / float(NTOK)
                if last:
                    q = q * float(NTOK)
                q_ref[c] = q
                return rs_acc + jnp.sum(q, axis=1, keepdims=True)
            rs = lax.fori_loop(0, NCHUNK, col_step,
                               jnp.zeros((KCB, 1), jnp.float32))

        def arg_step(c, _):
            q = q_ref[c]
            mx = jnp.max(q, axis=0, keepdims=True)
            io = lax.broadcasted_iota(jnp.int32, (KCB, CW), 0)
            idc = jnp.min(jnp.where(q == mx, io, KCB), axis=0)
            idx_ref[c] = idc.reshape(1, CW)
            return 0
        lax.fori_loop(0, NCHUNK, arg_step, 0)


def _assign(x, emb0, w, b2):
    return pl.pallas_call(
        _assign_body,
        grid=(NCHUNK,),
        in_specs=[
            pl.BlockSpec((DTOK, CW), lambda i: (0, i)),
            pl.BlockSpec((KCB, DTOK), lambda i: (0, 0)),
            pl.BlockSpec((DTOK, DTOK), lambda i: (0, 0)),
            pl.BlockSpec((1, DTOK), lambda i: (0, 0)),
        ],
        out_specs=[
            pl.BlockSpec((NCHUNK, 1, CW), lambda i: (0, 0, 0)),
            pl.BlockSpec((KCB, DTOK), lambda i: (0, 0)),
        ],
        out_shape=[
            jax.ShapeDtypeStruct((NCHUNK, 1, CW), jnp.int32),
            jax.ShapeDtypeStruct((KCB, DTOK), jnp.float32),
        ],
        scratch_shapes=[
            pltpu.VMEM((NCHUNK, KCB, CW), jnp.float32),
            pltpu.VMEM((NCHUNK, 1, CW), jnp.float32),
            pltpu.VMEM((KCB, DTOK), jnp.float32),
            pltpu.VMEM((KCB, 1), jnp.float32),
            pltpu.SMEM((2,), jnp.float32),
        ],
    )(x, emb0, w, b2)


NW = 32
BPW = NTOK // NW
GCH = 3
GW = BPW // GCH


def _sc_gather(table, idx3):
    mesh = plsc.VectorSubcoreMesh(core_axis_name="c", subcore_axis_name="s")

    @functools.partial(
        pl.kernel, mesh=mesh,
        out_type=jax.ShapeDtypeStruct((NTOK, DTOK), jnp.float32),
        scratch_types=[
            pltpu.VMEM((GCH, GW), jnp.int32),
            pltpu.VMEM((BPW, DTOK), jnp.float32),
            pltpu.SemaphoreType.DMA,
        ],
    )
    def k(table_hbm, idx_hbm, out_hbm, idx_v, rows_v, sem):
        wid = lax.axis_index("s") * 2 + lax.axis_index("c")
        pltpu.sync_copy(idx_hbm.at[wid], idx_v)
        cps = [pltpu.async_copy(table_hbm.at[idx_v.at[j]],
                                rows_v.at[pl.ds(j * GW, GW)], sem)
               for j in range(GCH)]
        for cp in cps:
            cp.wait()
        pltpu.sync_copy(rows_v, out_hbm.at[pl.ds(wid * BPW, BPW)])

    return k(table, idx3)


def _final_body(z_ref, zq_ref, zo_ref, ss_ref):
    i = pl.program_id(0)

    @pl.when(i == 0)
    def _init():
        ss_ref[0, 0] = 0.0

    zb = z_ref[0]
    zqt = zq_ref[0].T
    diff = zqt - zb
    zo_ref[0] = zb + diff
    ss_ref[0, 0] = ss_ref[0, 0] + jnp.sum(diff * diff)


def _finalize(z3, zq3):
    nb = z3.shape[0]
    hw = z3.shape[2]
    return pl.pallas_call(
        _final_body,
        grid=(nb,),
        in_specs=[
            pl.BlockSpec((1, DTOK, hw), lambda i: (i, 0, 0)),
            pl.BlockSpec((1, hw, DTOK), lambda i: (i, 0, 0)),
        ],
        out_specs=[
            pl.BlockSpec((1, DTOK, hw), lambda i: (i, 0, 0)),
            pl.BlockSpec(memory_space=pltpu.SMEM),
        ],
        out_shape=[
            jax.ShapeDtypeStruct((nb, DTOK, hw), jnp.float32),
            jax.ShapeDtypeStruct((1, 1), jnp.float32),
        ],
    )(z3, zq3)


def kernel(z, embedding, W, b):
    z = z.astype(jnp.float32)
    nb, nc, nh, nw = z.shape
    hw = nh * nw
    z3 = z.reshape(nb, nc, hw)
    x = z3.transpose(1, 0, 2).reshape(nc, nb * hw)

    idx12, table = _assign(x, embedding[0], W, b.reshape(1, DTOK))
    idx_flat = idx12.reshape(NTOK)

    zq = _sc_gather(table, idx_flat.reshape(NW, GCH, GW))

    zo3, ssum = _finalize(z3, zq.reshape(nb, hw, DTOK))

    m = ssum[0, 0] / float(nb * hw * DTOK)
    commitment = 0.25 * m
    codebook = m
    loss = commitment + codebook
    z_out = zo3.reshape(nb, nc, nh, nw)
    mei = idx_flat.reshape(nb, nh, nw)
    return (z_out, loss, commitment, codebook, mei)


# speedup vs baseline: 1.9871x; 1.9871x over previous
import functools

import jax
import jax.numpy as jnp
from jax import lax
from jax.experimental import pallas as pl
from jax.experimental.pallas import tpu as pltpu
from jax.experimental.pallas import tpu_sc as plsc

KCB = 1024
DTOK = 256
NTOK = 9216
NCHUNK = 12
CW = NTOK // NCHUNK
EPS = 10.0
NITER = 5


def _assign_body(x_ref, zt_ref, e_ref, w_ref, b_ref, idx_ref, emb_ref,
                 q_ref, cs_ref, y_ref, y2_ref, smem):
    i = pl.program_id(0)

    @pl.when(i == 0)
    def _init():
        y = lax.dot_general(e_ref[...], w_ref[...],
                            (((1,), (1,)), ((), ())),
                            preferred_element_type=jnp.float32)
        y = y + b_ref[...]
        y_ref[...] = y
        emb_ref[...] = y
        y2_ref[...] = jnp.sum(y * y, axis=1, keepdims=True)

    xb = x_ref[...]
    sc = lax.dot_general(y_ref[...], xb, (((1,), (0,)), ((), ())),
                         preferred_element_type=jnp.float32)
    ztb = zt_ref[...]
    x2 = jnp.sum(ztb * ztb, axis=1, keepdims=True).T
    d2 = (x2 + y2_ref[...]) - 2.0 * sc
    d = jnp.sqrt(jnp.maximum(d2, 0.0))
    q_ref[i] = d

    @pl.when(i == NCHUNK - 1)
    def _final():
        kn = float(KCB * NTOK)

        def s1_step(c, carry):
            s1c, mnc = carry
            return s1c + jnp.sum(q_ref[c]), jnp.minimum(mnc, jnp.min(q_ref[c]))
        s1, mn = lax.fori_loop(0, NCHUNK, s1_step, (0.0, jnp.inf))
        mean = s1 / kn

        def s2_oct(j, acc):
            def s2_chunk(c, acc):
                t = q_ref[c, pl.ds(j * 8, 8), :] - mean
                t = t * t
                for l in range(CW // 128):
                    acc = acc + t[:, l * 128:(l + 1) * 128]
                return acc
            return lax.fori_loop(0, NCHUNK, s2_chunk, acc)
        s2acc = lax.fori_loop(0, KCB // 8, s2_oct,
                              jnp.zeros((8, 128), jnp.float32))
        t8 = jnp.sum(s2acc.T, axis=0, keepdims=True)
        s2 = jnp.sum(t8)
        std = jnp.sqrt(s2 / (kn - 1.0))
        denom = std + 1e-8
        dmin = (mn - mean) / denom

        def exp_step(c, acc):
            dn = (q_ref[c] - mean) / denom - dmin
            q = jnp.exp(-dn * EPS)
            q_ref[c] = q
            return acc + jnp.sum(q)
        s = lax.fori_loop(0, NCHUNK, exp_step, 0.0)

        def _acc_tiles(q, acc):
            for l in range(CW // 128):
                acc = acc + q[:, l * 128:(l + 1) * 128]
            return acc

        def _lane_tree(acc):
            return jnp.sum(acc.T, axis=0, keepdims=True).T

        def sdiv_step(c, acc):
            q = q_ref[c] / (s + 1e-8)
            q_ref[c] = q
            return _acc_tiles(q, acc)
        acc = lax.fori_loop(0, NCHUNK, sdiv_step,
                            jnp.zeros((KCB, 128), jnp.float32))
        rs = _lane_tree(acc)

        for it in range(NITER):
            def row_step(c, _):
                q = q_ref[c] / (rs + 1e-8)
                q = q / float(KCB)
                q_ref[c] = q
                cs_ref[c] = jnp.sum(q, axis=0, keepdims=True)
                return 0
            lax.fori_loop(0, NCHUNK, row_step, 0)

            last = it == NITER - 1

            def col_step(c, acc):
                q = q_ref[c] / (cs_ref[c] + 1e-8)
                q = q / float(NTOK)
                if last:
                    q = q * float(NTOK)
                q_ref[c] = q
                return _acc_tiles(q, acc)
            acc = lax.fori_loop(0, NCHUNK, col_step,
                                jnp.zeros((KCB, 128), jnp.float32))
            rs = _lane_tree(acc)

        def arg_step(c, _):
            q = q_ref[c]
            mx = jnp.max(q, axis=0, keepdims=True)
            io = lax.broadcasted_iota(jnp.int32, (KCB, CW), 0)
            idc = jnp.min(jnp.where(q == mx, io, KCB), axis=0)
            idx_ref[c] = idc.reshape(1, CW)
            return 0
        lax.fori_loop(0, NCHUNK, arg_step, 0)


def _assign(x, zt, emb0, w, b2):
    return pl.pallas_call(
        _assign_body,
        grid=(NCHUNK,),
        in_specs=[
            pl.BlockSpec((DTOK, CW), lambda i: (0, i)),
            pl.BlockSpec((CW, DTOK), lambda i: (i, 0)),
            pl.BlockSpec((KCB, DTOK), lambda i: (0, 0)),
            pl.BlockSpec((DTOK, DTOK), lambda i: (0, 0)),
            pl.BlockSpec((1, DTOK), lambda i: (0, 0)),
        ],
        out_specs=[
            pl.BlockSpec((NCHUNK, 1, CW), lambda i: (0, 0, 0)),
            pl.BlockSpec((KCB, DTOK), lambda i: (0, 0)),
        ],
        out_shape=[
            jax.ShapeDtypeStruct((NCHUNK, 1, CW), jnp.int32),
            jax.ShapeDtypeStruct((KCB, DTOK), jnp.float32),
        ],
        scratch_shapes=[
            pltpu.VMEM((NCHUNK, KCB, CW), jnp.float32),
            pltpu.VMEM((NCHUNK, 1, CW), jnp.float32),
            pltpu.VMEM((KCB, DTOK), jnp.float32),
            pltpu.VMEM((KCB, 1), jnp.float32),
            pltpu.SMEM((2,), jnp.float32),
        ],
    )(x, zt, emb0, w, b2)


NW = 32
BPW = NTOK // NW
GCH = 3
GW = BPW // GCH


def _sc_gather(table, idx3):
    mesh = plsc.VectorSubcoreMesh(core_axis_name="c", subcore_axis_name="s")

    @functools.partial(
        pl.kernel, mesh=mesh,
        out_type=jax.ShapeDtypeStruct((NTOK, DTOK), jnp.float32),
        scratch_types=[
            pltpu.VMEM((GCH, GW), jnp.int32),
            pltpu.VMEM((BPW, DTOK), jnp.float32),
            pltpu.SemaphoreType.DMA,
        ],
    )
    def k(table_hbm, idx_hbm, out_hbm, idx_v, rows_v, sem):
        wid = lax.axis_index("s") * 2 + lax.axis_index("c")
        pltpu.sync_copy(idx_hbm.at[wid], idx_v)
        cps = [pltpu.async_copy(table_hbm.at[idx_v.at[j]],
                                rows_v.at[pl.ds(j * GW, GW)], sem)
               for j in range(GCH)]
        for cp in cps:
            cp.wait()
        pltpu.sync_copy(rows_v, out_hbm.at[pl.ds(wid * BPW, BPW)])

    return k(table, idx3)


def _final_body(z_ref, zq_ref, zo_ref, ss_ref):
    i = pl.program_id(0)

    @pl.when(i == 0)
    def _init():
        ss_ref[0, 0] = 0.0

    zb = z_ref[0]
    zqt = zq_ref[0].T
    diff = zqt - zb
    zo_ref[0] = zb + diff
    ss_ref[0, 0] = ss_ref[0, 0] + jnp.sum(diff * diff)


def _finalize(z3, zq3):
    nb = z3.shape[0]
    hw = z3.shape[2]
    return pl.pallas_call(
        _final_body,
        grid=(nb,),
        in_specs=[
            pl.BlockSpec((1, DTOK, hw), lambda i: (i, 0, 0)),
            pl.BlockSpec((1, hw, DTOK), lambda i: (i, 0, 0)),
        ],
        out_specs=[
            pl.BlockSpec((1, DTOK, hw), lambda i: (i, 0, 0)),
            pl.BlockSpec(memory_space=pltpu.SMEM),
        ],
        out_shape=[
            jax.ShapeDtypeStruct((nb, DTOK, hw), jnp.float32),
            jax.ShapeDtypeStruct((1, 1), jnp.float32),
        ],
    )(z3, zq3)


def kernel(z, embedding, W, b):
    z = z.astype(jnp.float32)
    nb, nc, nh, nw = z.shape
    hw = nh * nw
    z3 = z.reshape(nb, nc, hw)
    x = z3.transpose(1, 0, 2).reshape(nc, nb * hw)
    zt = z3.transpose(0, 2, 1).reshape(nb * hw, nc)

    idx12, table = _assign(x, zt, embedding[0], W, b.reshape(1, DTOK))
    idx_flat = idx12.reshape(NTOK)

    zq = _sc_gather(table, idx_flat.reshape(NW, GCH, GW))

    zo3, ssum = _finalize(z3, zq.reshape(nb, hw, DTOK))

    m = ssum[0, 0] / float(nb * hw * DTOK)
    commitment = 0.25 * m
    codebook = m
    loss = commitment + codebook
    z_out = zo3.reshape(nb, nc, nh, nw)
    mei = idx_flat.reshape(nb, nh, nw)
    return (z_out, loss, commitment, codebook, mei)
